# Initial kernel scaffold; baseline (speedup 1.0000x reference)
import jax
import jax.numpy as jnp
from jax.experimental import pallas as pl


def kernel(x, edge_index, W1, b1, W2, b2, W3, b3):
    raise NotImplementedError("write your pallas kernel here")


# speedup vs baseline: 8.3077x; 8.3077x over previous
import functools

import jax
import jax.numpy as jnp
from jax import lax
from jax.experimental import pallas as pl
from jax.experimental.pallas import tpu as pltpu
from jax.experimental.pallas import tpu_sc as plsc

N = 10000
E = 320000
F = 128
CH = 128
NC = 2
NS = 16
NPAD = 10112
ROWS_PER_TILE = NPAD // NS
DUMMY = N

CHUNKS_A = 2560
CPT_A = CHUNKS_A // NS
HALF_E = E // 2
CHUNKS_3H = 1280
CPT_3 = CHUNKS_3H // NS

_mesh = plsc.VectorSubcoreMesh(core_axis_name="c", subcore_axis_name="s",
                               num_cores=NC, num_subcores=NS)


SEC = 16


def _make_agg(cpt, core_stride):
    nsec = cpt // SEC

    def body(t0, t1, srcc, dstc, out0, out1,
             src_v, dst_v, buf0, buf1, acc, g0, g1):
        c = lax.axis_index("c")
        s = lax.axis_index("s")
        base = c * core_stride + s * cpt

        zero16 = jnp.zeros((16,), jnp.float32)

        @pl.loop(0, CH)
        def _(r):
            for k in range(F // 16):
                buf0[r, pl.ds(k * 16, 16)] = zero16

        row0 = s * ROWS_PER_TILE
        for j in range(ROWS_PER_TILE // 128):
            pltpu.sync_copy(buf0.at[pl.ds(0, 128)],
                            acc.at[pl.ds(row0 + j * 128, 128)])
        _rem = ROWS_PER_TILE % 128
        pltpu.sync_copy(buf0.at[pl.ds(0, _rem)],
                        acc.at[pl.ds(row0 + ROWS_PER_TILE - _rem, _rem)])
        plsc.subcore_barrier()

        def run(tab, outr):
            @pl.loop(0, nsec)
            def _(sec):
                srow = base + sec * SEC
                pltpu.sync_copy(srcc.at[pl.ds(srow, SEC)], src_v)
                pltpu.sync_copy(dstc.at[pl.ds(srow, SEC)], dst_v)
                pltpu.async_copy(tab.at[src_v.at[0]], buf0, g0)

                @pl.loop(0, SEC, step=2)
                def _(j):
                    pltpu.async_copy(tab.at[src_v.at[j + 1]], buf1, g1)
                    pltpu.make_async_copy(tab.at[src_v.at[j]], buf0,
                                          g0).wait()
                    pltpu.sync_copy(buf0, acc.at[dst_v.at[j]], add=True)

                    @pl.when(j + 2 < SEC)
                    def _():
                        pltpu.async_copy(tab.at[src_v.at[j + 2]], buf0, g0)

                    pltpu.make_async_copy(tab.at[src_v.at[j + 1]], buf1,
                                          g1).wait()
                    pltpu.sync_copy(buf1, acc.at[dst_v.at[j + 1]], add=True)

            plsc.subcore_barrier()
            last = (NS - 1) * ROWS_PER_TILE
            nlast = N - last

            @pl.when(s < NS - 1)
            def _():
                pltpu.sync_copy(acc.at[pl.ds(row0, ROWS_PER_TILE)],
                                outr.at[pl.ds(row0, ROWS_PER_TILE)])

            @pl.when(s == NS - 1)
            def _():
                pltpu.sync_copy(acc.at[pl.ds(last, nlast)],
                                outr.at[pl.ds(last, nlast)])

        @pl.when(c == 0)
        def _():
            run(t0, out0)

        @pl.when(c == 1)
        def _():
            run(t1, out1)

    out_t = jax.ShapeDtypeStruct((N, F), jnp.float32)
    return pl.kernel(
        body,
        out_type=(out_t, out_t),
        mesh=_mesh,
        scratch_types=[
            pltpu.VMEM((SEC, CH), jnp.int32),
            pltpu.VMEM((SEC, CH), jnp.int32),
            pltpu.VMEM((CH, F), jnp.float32),
            pltpu.VMEM((CH, F), jnp.float32),
            pltpu.VMEM_SHARED((NPAD, F), jnp.float32),
            pltpu.SemaphoreType.DMA,
            pltpu.SemaphoreType.DMA,
        ],
        name="gcn_sc_agg",
    )


_agg12 = _make_agg(CPT_A, 0)
_agg3 = _make_agg(CPT_3, CHUNKS_3H)

E_PER_W = E // (NC * NS)


def _deg_body(dst_hbm, outp, dst_v, accd):
    c = lax.axis_index("c")
    s = lax.axis_index("s")
    wid = s * NC + c
    pltpu.sync_copy(dst_hbm.at[pl.ds(wid * E_PER_W, E_PER_W)], dst_v)
    zero16 = jnp.zeros((16,), jnp.float32)

    @pl.loop(0, NPAD // 16)
    def _(r):
        accd[pl.ds(r * 16, 16)] = zero16

    ones16 = jnp.ones((16,), jnp.float32)

    @pl.loop(0, E_PER_W // 16)
    def _(i):
        idx = dst_v[pl.ds(i * 16, 16)]
        plsc.addupdate_scatter(accd, [idx], ones16)

    pltpu.sync_copy(accd, outp.at[wid])


_deg = pl.kernel(
    _deg_body,
    out_type=jax.ShapeDtypeStruct((NC * NS, NPAD), jnp.float32),
    mesh=_mesh,
    scratch_types=[
        pltpu.VMEM((E_PER_W,), jnp.int32),
        pltpu.VMEM((NPAD,), jnp.float32),
    ],
    compiler_params=pltpu.CompilerParams(needs_layout_passes=False),
    name="gcn_sc_deg",
)


_BR = 1000
_GRID = N // _BR


def _degred_body(part_ref, dis_ref):
    deg = 1.0 + jnp.sum(part_ref[...], axis=0)
    dis_ref[...] = lax.rsqrt(deg)[:, None]


_degred = pl.pallas_call(
    _degred_body,
    grid=(1,),
    in_specs=[pl.BlockSpec((NC * NS, NPAD), lambda i: (0, 0))],
    out_specs=pl.BlockSpec((NPAD, 1), lambda i: (0, 0)),
    out_shape=jax.ShapeDtypeStruct((NPAD, 1), jnp.float32),
)


def _prep_body(x_ref, w_ref, dis_ref, hs0_ref, hs1_ref):
    dis = dis_ref[...]
    h = jnp.dot(x_ref[...], w_ref[...], preferred_element_type=jnp.float32)
    hs = h * dis
    hs0_ref[...] = hs[:, :F]
    hs1_ref[...] = hs[:, F:]


_prep = pl.pallas_call(
    _prep_body,
    grid=(_GRID,),
    in_specs=[
        pl.BlockSpec((_BR, 128), lambda i: (i, 0)),
        pl.BlockSpec((128, 256), lambda i: (0, 0)),
        pl.BlockSpec((_BR, 1), lambda i: (i, 0)),
    ],
    out_specs=[
        pl.BlockSpec((_BR, F), lambda i: (i, 0)),
        pl.BlockSpec((_BR, F), lambda i: (i, 0)),
    ],
    out_shape=[
        jax.ShapeDtypeStruct((N, F), jnp.float32),
        jax.ShapeDtypeStruct((N, F), jnp.float32),
    ],
)


def _make_mid(f_out, split):
    def body(a0_ref, a1_ref, hs0_ref, hs1_ref, dis_ref, b_ref, w_ref, *outs):
        dis = dis_ref[...]
        pre = jnp.concatenate(
            [a0_ref[...] + hs0_ref[...], a1_ref[...] + hs1_ref[...]], axis=1)
        act = jnp.maximum(pre * dis + b_ref[...], 0.0)
        h = jnp.dot(act, w_ref[...], preferred_element_type=jnp.float32)
        hs = h * dis
        if split:
            outs[0][...] = hs[:, :F]
            outs[1][...] = hs[:, F:]
        else:
            outs[0][...] = hs

    if split:
        out_shape = [jax.ShapeDtypeStruct((N, F), jnp.float32)] * 2
        out_specs = [pl.BlockSpec((_BR, F), lambda i: (i, 0))] * 2
    else:
        out_shape = [jax.ShapeDtypeStruct((N, f_out), jnp.float32)]
        out_specs = [pl.BlockSpec((_BR, f_out), lambda i: (i, 0))]
    return pl.pallas_call(
        body,
        grid=(_GRID,),
        in_specs=[
            pl.BlockSpec((_BR, F), lambda i: (i, 0)),
            pl.BlockSpec((_BR, F), lambda i: (i, 0)),
            pl.BlockSpec((_BR, F), lambda i: (i, 0)),
            pl.BlockSpec((_BR, F), lambda i: (i, 0)),
            pl.BlockSpec((_BR, 1), lambda i: (i, 0)),
            pl.BlockSpec((1, 256), lambda i: (0, 0)),
            pl.BlockSpec((256, f_out), lambda i: (0, 0)),
        ],
        out_specs=out_specs,
        out_shape=out_shape,
    )


_mid_split = _make_mid(256, True)
_mid3 = _make_mid(128, False)


def _final_body(a0_ref, a1_ref, hs_ref, dis_ref, b_ref, out_ref):
    out_ref[...] = ((a0_ref[...] + a1_ref[...] + hs_ref[...]) * dis_ref[...]
                    + b_ref[...])


_final = pl.pallas_call(
    _final_body,
    grid=(_GRID,),
    in_specs=[
        pl.BlockSpec((_BR, F), lambda i: (i, 0)),
        pl.BlockSpec((_BR, F), lambda i: (i, 0)),
        pl.BlockSpec((_BR, F), lambda i: (i, 0)),
        pl.BlockSpec((_BR, 1), lambda i: (i, 0)),
        pl.BlockSpec((1, 128), lambda i: (0, 0)),
    ],
    out_specs=pl.BlockSpec((_BR, F), lambda i: (i, 0)),
    out_shape=jax.ShapeDtypeStruct((N, F), jnp.float32),
)


def _chunk_pad(a, total, fill):
    pad = jnp.full((total - a.shape[0],), fill, jnp.int32)
    return jnp.concatenate([a, pad]).reshape(-1, CH)


def kernel(x, edge_index, W1, b1, W2, b2, W3, b3):
    src = edge_index[0].astype(jnp.int32)
    dst = edge_index[1].astype(jnp.int32)

    src_a = _chunk_pad(src, CHUNKS_A * CH, 0)
    dst_a = _chunk_pad(dst, CHUNKS_A * CH, DUMMY)
    src_3 = jnp.concatenate([
        _chunk_pad(src[:HALF_E], CHUNKS_3H * CH, 0),
        _chunk_pad(src[HALF_E:], CHUNKS_3H * CH, 0),
    ])
    dst_3 = jnp.concatenate([
        _chunk_pad(dst[:HALF_E], CHUNKS_3H * CH, DUMMY),
        _chunk_pad(dst[HALF_E:], CHUNKS_3H * CH, DUMMY),
    ])

    partials = _deg(dst)
    dis = _degred(partials)[:N]
    hs10, hs11 = _prep(x, W1, dis)
    a10, a11 = _agg12(hs10, hs11, src_a, dst_a)
    hs20, hs21 = _mid_split(a10, a11, hs10, hs11, dis,
                            b1.reshape(1, 256), W2)
    a20, a21 = _agg12(hs20, hs21, src_a, dst_a)
    (hs3,) = _mid3(a20, a21, hs20, hs21, dis, b2.reshape(1, 256), W3)
    a30, a31 = _agg3(hs3, hs3, src_3, dst_3)
    return _final(a30, a31, hs3, dis, b3.reshape(1, 128))
